# baseline (speedup 1.0000x reference)
import jax
import jax.numpy as jnp
from jax import lax
from jax.experimental import pallas as pl
from jax.experimental.pallas import tpu as pltpu
from jax.experimental.pallas import tpu_sc as plsc

N_FACTORS = 64
BATCH = 16384
CHUNK = 128


def _mf_kernel(uid_hbm, iid_hbm, uemb_hbm, iemb_hbm, ubias_hbm, ibias_hbm,
               gbias_hbm, out_hbm,
               uidx_v, iidx_v, urows_v, irows_v, ub_v, ib_v, gb_v, out_v,
               usem, isem, bsem):
    info = plsc.get_sparse_core_info()
    nc = info.num_cores
    wid = lax.axis_index("s") * nc + lax.axis_index("c")
    n_chunks = uidx_v.shape[0]
    b_per_w = n_chunks * CHUNK
    base = wid * b_per_w

    for j in range(n_chunks):
        pltpu.sync_copy(uid_hbm.at[pl.ds(base + j * CHUNK, CHUNK)],
                        uidx_v.at[j])
        pltpu.sync_copy(iid_hbm.at[pl.ds(base + j * CHUNK, CHUNK)],
                        iidx_v.at[j])
    pltpu.sync_copy(gbias_hbm, gb_v)

    bias_copies = []
    for j in range(n_chunks):
        sl = pl.ds(j * CHUNK, CHUNK)
        bias_copies.append(pltpu.async_copy(ubias_hbm.at[uidx_v.at[j]],
                                            ub_v.at[sl], bsem))
        bias_copies.append(pltpu.async_copy(ibias_hbm.at[iidx_v.at[j]],
                                            ib_v.at[sl], bsem))

    gvec = gb_v[...]
    lanes = lax.iota(jnp.int32, 16)

    u_copies = []
    for j in range(n_chunks):
        sl = pl.ds(j * CHUNK, CHUNK)
        u_copies.append(pltpu.async_copy(uemb_hbm.at[uidx_v.at[j]],
                                         urows_v.at[sl], usem))

    def fire_item(j):
        return pltpu.async_copy(iemb_hbm.at[iidx_v.at[j]],
                                irows_v.at[pl.ds((j % 2) * CHUNK, CHUNK)],
                                isem)

    i_copies = {0: fire_item(0), 1: fire_item(1)}

    for j in range(n_chunks):
        u_copies[j].wait()
        i_copies[j].wait()
        if j == 0:
            for c in bias_copies:
                c.wait()

        def group_body(g, _, j=j):
            off = j * CHUNK + g * 16
            rows = off + lanes
            lrows = (j % 2) * CHUNK + g * 16 + lanes
            acc = ub_v[pl.ds(off, 16)] + ib_v[pl.ds(off, 16)] + gvec
            for d in range(N_FACTORS):
                col = jnp.full((16,), d, jnp.int32)
                u = plsc.load_gather(urows_v, [rows, col])
                v = plsc.load_gather(irows_v, [lrows, col])
                acc = acc + u * v
            out_v[pl.ds(off, 16)] = acc
            return _

        lax.fori_loop(0, CHUNK // 16, group_body, 0, unroll=False)
        if j + 2 < n_chunks:
            i_copies[j + 2] = fire_item(j + 2)

    pltpu.sync_copy(out_v, out_hbm.at[pl.ds(base, b_per_w)])


def kernel(user_ids, item_ids, user_embedding, item_embedding, user_bias,
           item_bias, global_bias):
    nw = 32
    b_per_w = BATCH // nw
    n_chunks = b_per_w // CHUNK

    uid = user_ids.astype(jnp.int32)
    iid = item_ids.astype(jnp.int32)
    ue2 = jnp.pad(user_embedding, ((0, 0), (0, CHUNK - N_FACTORS)))
    ie2 = jnp.pad(item_embedding, ((0, 0), (0, CHUNK - N_FACTORS)))
    ub = user_bias.reshape(-1)
    ib = item_bias.reshape(-1)
    gb = jnp.broadcast_to(global_bias.astype(jnp.float32), (16,))

    mesh = plsc.VectorSubcoreMesh(core_axis_name="c", subcore_axis_name="s")
    f = pl.kernel(
        _mf_kernel,
        mesh=mesh,
        compiler_params=pltpu.CompilerParams(needs_layout_passes=False,
                                             use_tc_tiling_on_sc=True),
        out_type=jax.ShapeDtypeStruct((BATCH,), jnp.float32),
        scratch_types=[
            pltpu.VMEM((n_chunks, CHUNK), jnp.int32),
            pltpu.VMEM((n_chunks, CHUNK), jnp.int32),
            pltpu.VMEM((b_per_w, CHUNK), jnp.float32),
            pltpu.VMEM((2 * CHUNK, CHUNK), jnp.float32),
            pltpu.VMEM((b_per_w,), jnp.float32),
            pltpu.VMEM((b_per_w,), jnp.float32),
            pltpu.VMEM((16,), jnp.float32),
            pltpu.VMEM((b_per_w,), jnp.float32),
            pltpu.SemaphoreType.DMA,
            pltpu.SemaphoreType.DMA,
            pltpu.SemaphoreType.DMA,
        ],
    )
    return f(uid, iid, ue2, ie2, ub, ib, gb)
